# baseline (speedup 1.0000x reference)
import functools

import jax
import jax.numpy as jnp
from jax import lax
from jax.experimental import pallas as pl
from jax.experimental.pallas import tpu as pltpu
from jax.experimental.pallas import tpu_sc as plsc


def _sc_convert(x_flat, B, L):
    n_rows, d = x_flat.shape
    mesh = plsc.VectorSubcoreMesh(core_axis_name="c", subcore_axis_name="s")
    info = plsc.get_sparse_core_info()
    n_workers = info.num_cores * info.num_subcores
    rows_per_w = n_rows // n_workers

    @functools.partial(
        pl.kernel,
        mesh=mesh,
        out_type=[
            jax.ShapeDtypeStruct((n_rows, d), x_flat.dtype),
            jax.ShapeDtypeStruct((32,), jnp.int32),
        ],
        scratch_types=[pltpu.VMEM((32,), jnp.int32)],
    )
    def k(x_hbm, val_hbm, off_hbm, off_v):
        cid = lax.axis_index("c")
        sid = lax.axis_index("s")
        wid = sid * info.num_cores + cid
        base = wid * rows_per_w
        pltpu.sync_copy(
            x_hbm.at[pl.ds(base, rows_per_w)],
            val_hbm.at[pl.ds(base, rows_per_w)],
        )

        @pl.when(jnp.logical_and(cid == 0, sid == 0))
        def _():
            lane = lax.iota(jnp.int32, 16)
            off_v[pl.ds(0, 16)] = lane * L
            off_v[pl.ds(16, 16)] = (lane + 16) * L
            pltpu.sync_copy(off_v, off_hbm)

    return k(x_flat)


def kernel(X):
    B, L, d = X.shape
    x_flat = X.reshape(B * L, d)
    values, offs = _sc_convert(x_flat, B, L)
    return (values, offs[: B + 1])


# speedup vs baseline: 48.5184x; 48.5184x over previous
import jax
import jax.numpy as jnp
from jax.experimental import pallas as pl
from jax.experimental.pallas import tpu as pltpu

_CHUNKS = 64
_NBUF = 8
_LOOKAHEAD = 4


def _body(x_hbm, o_hbm, off_ref, buf, in_sems, out_sems):
    n_rows = x_hbm.shape[0]
    b = off_ref.shape[0] - 1
    seq_len = n_rows // b
    for i in range(b + 1):
        off_ref[i] = i * seq_len

    rows = n_rows // _CHUNKS
    ins = [
        pltpu.make_async_copy(
            x_hbm.at[pl.ds(k * rows, rows)], buf.at[k % _NBUF],
            in_sems.at[k % _NBUF],
        )
        for k in range(_CHUNKS)
    ]
    outs = [
        pltpu.make_async_copy(
            buf.at[k % _NBUF], o_hbm.at[pl.ds(k * rows, rows)],
            out_sems.at[k % _NBUF],
        )
        for k in range(_CHUNKS)
    ]
    for k in range(_LOOKAHEAD):
        ins[k].start()
    for k in range(_CHUNKS):
        if k >= _LOOKAHEAD:
            outs[k - _LOOKAHEAD].wait()
        if k + _LOOKAHEAD < _CHUNKS:
            ins[k + _LOOKAHEAD].start()
        ins[k].wait()
        outs[k].start()
    for k in range(_CHUNKS - _LOOKAHEAD, _CHUNKS):
        outs[k].wait()


def kernel(X):
    B, L, d = X.shape
    x_flat = X.reshape(B * L, d)
    n_rows = B * L
    rows = n_rows // _CHUNKS
    values, offsets = pl.pallas_call(
        _body,
        in_specs=[pl.BlockSpec(memory_space=pl.ANY)],
        out_specs=[
            pl.BlockSpec(memory_space=pl.ANY),
            pl.BlockSpec(memory_space=pltpu.SMEM),
        ],
        out_shape=[
            jax.ShapeDtypeStruct((n_rows, d), x_flat.dtype),
            jax.ShapeDtypeStruct((B + 1,), jnp.int32),
        ],
        scratch_shapes=[
            pltpu.VMEM((_NBUF, rows, d), x_flat.dtype),
            pltpu.SemaphoreType.DMA((_NBUF,)),
            pltpu.SemaphoreType.DMA((_NBUF,)),
        ],
    )(x_flat)
    return (values, offsets)


# speedup vs baseline: 48.5528x; 1.0007x over previous
import jax
import jax.numpy as jnp
from jax.experimental import pallas as pl
from jax.experimental.pallas import tpu as pltpu

_CHUNKS = 32
_NBUF = 4
_LOOKAHEAD = 2


def _body(x_hbm, o_hbm, off_ref, buf, in_sems, out_sems):
    n_rows = x_hbm.shape[0]
    b = off_ref.shape[0] - 1
    seq_len = n_rows // b
    for i in range(b + 1):
        off_ref[i] = i * seq_len

    rows = n_rows // _CHUNKS
    ins = [
        pltpu.make_async_copy(
            x_hbm.at[pl.ds(k * rows, rows)], buf.at[k % _NBUF],
            in_sems.at[k % _NBUF],
        )
        for k in range(_CHUNKS)
    ]
    outs = [
        pltpu.make_async_copy(
            buf.at[k % _NBUF], o_hbm.at[pl.ds(k * rows, rows)],
            out_sems.at[k % _NBUF],
        )
        for k in range(_CHUNKS)
    ]
    for k in range(_LOOKAHEAD):
        ins[k].start()
    for k in range(_CHUNKS):
        if k >= _LOOKAHEAD:
            outs[k - _LOOKAHEAD].wait()
        if k + _LOOKAHEAD < _CHUNKS:
            ins[k + _LOOKAHEAD].start()
        ins[k].wait()
        outs[k].start()
    for k in range(_CHUNKS - _LOOKAHEAD, _CHUNKS):
        outs[k].wait()


def kernel(X):
    B, L, d = X.shape
    x_flat = X.reshape(B * L, d)
    n_rows = B * L
    rows = n_rows // _CHUNKS
    values, offsets = pl.pallas_call(
        _body,
        in_specs=[pl.BlockSpec(memory_space=pl.ANY)],
        out_specs=[
            pl.BlockSpec(memory_space=pl.ANY),
            pl.BlockSpec(memory_space=pltpu.SMEM),
        ],
        out_shape=[
            jax.ShapeDtypeStruct((n_rows, d), x_flat.dtype),
            jax.ShapeDtypeStruct((B + 1,), jnp.int32),
        ],
        scratch_shapes=[
            pltpu.VMEM((_NBUF, rows, d), x_flat.dtype),
            pltpu.SemaphoreType.DMA((_NBUF,)),
            pltpu.SemaphoreType.DMA((_NBUF,)),
        ],
    )(x_flat)
    return (values, offsets)
